# baseline (speedup 1.0000x reference)
import jax
import jax.numpy as jnp
from jax import lax
from jax.experimental import pallas as pl
from jax.experimental.pallas import tpu as pltpu
from jax.experimental.pallas import tpu_sc as plsc

_POOL = 7
_NUM_ROIS = 300
_H = 64
_W = 64
_C = 512
_NUNITS = _NUM_ROIS * _POOL

_info = plsc.get_sparse_core_info()
_NC = _info.num_cores
_NS = _info.num_subcores
_NW = _NC * _NS
_CV = _C // 16
_UPW = -(-_NUNITS // _NW)
_NPAIR = (_UPW + 2) // 2


def _body(img_hbm, rois_hbm, out_hbm, rois_v,
          idx0, idx1, wrow0, wrow1, rows0, rows1, out0, out1,
          gsem0, gsem1, wsem0, wsem1):
  idx = (idx0, idx1)
  wrow = (wrow0, wrow1)
  rows = (rows0, rows1)
  outv = (out0, out1)
  gsem = (gsem0, gsem1)
  wsem = (wsem0, wsem1)

  wid = lax.axis_index("s") * _NC + lax.axis_index("c")

  pltpu.sync_copy(rois_hbm, rois_v)

  lanes = lax.iota(jnp.int32, 16)
  pxv = jnp.minimum(jnp.where(lanes < 7, lanes, lanes - 7), 6)
  grpb = lanes >= 7
  live = lanes < 14

  def full16(v):
    return jnp.full((16,), v, jnp.int32)

  def unit_of(j):
    return wid + j * _NW

  def fire(u, b):
    r = lax.div(u, _POOL)
    py = u - r * _POOL

    xv = plsc.load_gather(rois_v, [full16(4 * r)])
    yv = plsc.load_gather(rois_v, [full16(4 * r + 1)])
    wv = plsc.load_gather(rois_v, [full16(4 * r + 2)])
    hv = plsc.load_gather(rois_v, [full16(4 * r + 3)])

    ys = full16(py).astype(jnp.float32) * (hv.astype(jnp.float32)
                                           / float(_POOL))
    y0 = jnp.minimum(ys.astype(jnp.int32), hv - 1)
    y1 = jnp.minimum(y0 + 1, hv - 1)
    wy = ys - y0.astype(jnp.float32)

    xs = pxv.astype(jnp.float32) * (wv.astype(jnp.float32) / float(_POOL))
    x0 = jnp.minimum(xs.astype(jnp.int32), wv - 1)
    x1 = jnp.minimum(x0 + 1, wv - 1)
    wx = xs - x0.astype(jnp.float32)

    col = xv + jnp.where(grpb, x1, x0)
    wcol = jnp.where(live, jnp.where(grpb, wx, 1.0 - wx), 0.0)

    idx[b][pl.ds(0, 16)] = (yv + y0) * _W + col
    idx[b][pl.ds(16, 16)] = (yv + y1) * _W + col
    wrow[b][pl.ds(0, 16)] = (1.0 - wy) * wcol
    wrow[b][pl.ds(16, 16)] = wy * wcol

    pltpu.async_copy(img_hbm.at[idx[b]], rows[b], gsem[b])

  def blend(b):
    rv = rows[b]
    ov = outv[b]

    def do_px(px, _):
      wa = plsc.load_gather(wrow[b], [full16(px)])
      wb = plsc.load_gather(wrow[b], [full16(px + 7)])
      wc = plsc.load_gather(wrow[b], [full16(px + 16)])
      wd = plsc.load_gather(wrow[b], [full16(px + 23)])
      for v in range(_CV):
        sl = pl.ds(v * 16, 16)
        acc = (rv[px, sl] * wa + rv[px + 7, sl] * wb
               + rv[px + 16, sl] * wc + rv[px + 23, sl] * wd)
        ov[px, sl] = acc
      return 0

    lax.fori_loop(0, _POOL, do_px, 0)

  def write(u, b):
    pltpu.async_copy(outv[b], out_hbm.at[u], wsem[b])

  def wait_gather(b):
    pltpu.make_async_copy(img_hbm.at[idx[b]], rows[b], gsem[b]).wait()

  def wait_write(b):
    pltpu.make_async_copy(outv[b], out_hbm.at[0], wsem[b]).wait()

  fire(unit_of(0), 0)

  def pair_body(t, _):
    for b in (0, 1):
      j = 2 * t + b
      u = unit_of(j)
      un = unit_of(j + 1)

      @pl.when(un < _NUNITS)
      def _():
        fire(un, 1 - b)

      @pl.when(u < _NUNITS)
      def _():
        wait_gather(b)
        @pl.when(j >= 2)
        def _():
          wait_write(b)

        blend(b)
        write(u, b)

    return 0

  lax.fori_loop(0, _NPAIR, pair_body, 0)

  wait_write(0)
  wait_write(1)


_G = 10


def _depad_body(in_ref, out_ref):
  x = in_ref[...].reshape(_G, _POOL, 8, _C)
  out_ref[...] = x[:, :, :_POOL, :]


@jax.jit
def kernel(img, rois):
  img2 = img.reshape(_H * _W, _C)
  rflat = rois.reshape(-1).astype(jnp.int32)
  mesh = plsc.VectorSubcoreMesh(core_axis_name="c", subcore_axis_name="s")
  staged = pl.kernel(
      _body,
      mesh=mesh,
      compiler_params=pltpu.CompilerParams(needs_layout_passes=False),
      out_type=jax.ShapeDtypeStruct((_NUNITS, 8, _C), jnp.float32),
      scratch_types=[
          pltpu.VMEM((_NUM_ROIS * 4,), jnp.int32),
          pltpu.VMEM((32,), jnp.int32),
          pltpu.VMEM((32,), jnp.int32),
          pltpu.VMEM((32,), jnp.float32),
          pltpu.VMEM((32,), jnp.float32),
          pltpu.VMEM((32, _C), jnp.float32),
          pltpu.VMEM((32, _C), jnp.float32),
          pltpu.VMEM((8, _C), jnp.float32),
          pltpu.VMEM((8, _C), jnp.float32),
          pltpu.SemaphoreType.DMA,
          pltpu.SemaphoreType.DMA,
          pltpu.SemaphoreType.DMA,
          pltpu.SemaphoreType.DMA,
      ],
  )(img2, rflat)

  out = pl.pallas_call(
      _depad_body,
      grid=(_NUM_ROIS // _G,),
      in_specs=[pl.BlockSpec((_G * _POOL, 8, _C), lambda i: (i, 0, 0))],
      out_specs=pl.BlockSpec((_G, _POOL, _POOL, _C),
                             lambda i: (i, 0, 0, 0)),
      out_shape=jax.ShapeDtypeStruct((_NUM_ROIS, _POOL, _POOL, _C),
                                     jnp.float32),
  )(staged)
  return out[None]


# speedup vs baseline: 1.4064x; 1.4064x over previous
import jax
import jax.numpy as jnp
from jax import lax
from jax.experimental import pallas as pl
from jax.experimental.pallas import tpu as pltpu
from jax.experimental.pallas import tpu_sc as plsc

_POOL = 7
_NUM_ROIS = 300
_H = 64
_W = 64
_C = 512
_NPLANES = _NUM_ROIS * _POOL
_NUNITS = _NPLANES // 2

_info = plsc.get_sparse_core_info()
_NC = _info.num_cores
_NS = _info.num_subcores
_NW = _NC * _NS
_CV = _C // 16
_UPW = -(-_NUNITS // _NW)
_NPAIR = (_UPW + 2) // 2


def _body(img_hbm, rois_hbm, out_hbm, rois_v,
          idx0, idx1, wrow0, wrow1, rows0, rows1, out0, out1,
          gsem0, gsem1, wsem0, wsem1):
  idx = (idx0, idx1)
  wrow = (wrow0, wrow1)
  rows = (rows0, rows1)
  outv = (out0, out1)
  gsem = (gsem0, gsem1)
  wsem = (wsem0, wsem1)

  wid = lax.axis_index("s") * _NC + lax.axis_index("c")

  pltpu.sync_copy(rois_hbm, rois_v)

  lanes = lax.iota(jnp.int32, 16)
  pxv = jnp.minimum(jnp.where(lanes < 7, lanes, lanes - 7), 6)
  grpb = lanes >= 7
  live = lanes < 14

  def full16(v):
    return jnp.full((16,), v, jnp.int32)

  def unit_of(j):
    return wid + j * _NW

  def fire(u, b):
    for i in (0, 1):
      plane = 2 * u + i
      r = lax.div(plane, _POOL)
      py = plane - r * _POOL

      xv = plsc.load_gather(rois_v, [full16(4 * r)])
      yv = plsc.load_gather(rois_v, [full16(4 * r + 1)])
      wv = plsc.load_gather(rois_v, [full16(4 * r + 2)])
      hv = plsc.load_gather(rois_v, [full16(4 * r + 3)])

      ys = full16(py).astype(jnp.float32) * (hv.astype(jnp.float32)
                                             / float(_POOL))
      y0 = jnp.minimum(ys.astype(jnp.int32), hv - 1)
      y1 = jnp.minimum(y0 + 1, hv - 1)
      wy = ys - y0.astype(jnp.float32)

      xs = pxv.astype(jnp.float32) * (wv.astype(jnp.float32) / float(_POOL))
      x0 = jnp.minimum(xs.astype(jnp.int32), wv - 1)
      x1 = jnp.minimum(x0 + 1, wv - 1)
      wx = xs - x0.astype(jnp.float32)

      col = xv + jnp.where(grpb, x1, x0)
      wcol = jnp.where(live, jnp.where(grpb, wx, 1.0 - wx), 0.0)

      idx[b][pl.ds(32 * i, 16)] = (yv + y0) * _W + col
      idx[b][pl.ds(32 * i + 16, 16)] = (yv + y1) * _W + col
      wrow[b][pl.ds(32 * i, 16)] = (1.0 - wy) * wcol
      wrow[b][pl.ds(32 * i + 16, 16)] = wy * wcol

    pltpu.async_copy(img_hbm.at[idx[b]], rows[b], gsem[b])

  def blend(b):
    rv = rows[b]
    ov = outv[b]

    def do_px(px, _):
      for i in (0, 1):
        o = 32 * i
        wa = plsc.load_gather(wrow[b], [full16(px + o)])
        wb = plsc.load_gather(wrow[b], [full16(px + o + 7)])
        wc = plsc.load_gather(wrow[b], [full16(px + o + 16)])
        wd = plsc.load_gather(wrow[b], [full16(px + o + 23)])
        for v in range(_CV):
          sl = pl.ds(v * 16, 16)
          acc = (rv[px + o, sl] * wa + rv[px + o + 7, sl] * wb
                 + rv[px + o + 16, sl] * wc + rv[px + o + 23, sl] * wd)
          ov[i, px, sl] = acc
      return 0

    lax.fori_loop(0, _POOL, do_px, 0)

  def write(u, b):
    pltpu.async_copy(outv[b], out_hbm.at[pl.ds(2 * u, 2)], wsem[b])

  def wait_gather(b):
    pltpu.make_async_copy(img_hbm.at[idx[b]], rows[b], gsem[b]).wait()

  def wait_write(b):
    pltpu.make_async_copy(outv[b], out_hbm.at[pl.ds(0, 2)], wsem[b]).wait()

  fire(unit_of(0), 0)

  def pair_body(t, _):
    for b in (0, 1):
      j = 2 * t + b
      u = unit_of(j)
      un = unit_of(j + 1)

      @pl.when(un < _NUNITS)
      def _():
        fire(un, 1 - b)

      @pl.when(u < _NUNITS)
      def _():
        wait_gather(b)
        @pl.when(j >= 2)
        def _():
          wait_write(b)

        blend(b)
        write(u, b)

    return 0

  lax.fori_loop(0, _NPAIR, pair_body, 0)

  wait_write(0)
  wait_write(1)


_G = 10


def _depad_body(in_ref, out_ref):
  x = in_ref[...].reshape(_G, _POOL, 8, _C)
  out_ref[0] = x[:, :, :_POOL, :]


@jax.jit
def kernel(img, rois):
  img2 = img.reshape(_H * _W, _C)
  rflat = rois.reshape(-1).astype(jnp.int32)
  mesh = plsc.VectorSubcoreMesh(core_axis_name="c", subcore_axis_name="s")
  staged = pl.kernel(
      _body,
      mesh=mesh,
      compiler_params=pltpu.CompilerParams(needs_layout_passes=False),
      out_type=jax.ShapeDtypeStruct((_NPLANES, 8, _C), jnp.float32),
      scratch_types=[
          pltpu.VMEM((_NUM_ROIS * 4,), jnp.int32),
          pltpu.VMEM((64,), jnp.int32),
          pltpu.VMEM((64,), jnp.int32),
          pltpu.VMEM((64,), jnp.float32),
          pltpu.VMEM((64,), jnp.float32),
          pltpu.VMEM((64, _C), jnp.float32),
          pltpu.VMEM((64, _C), jnp.float32),
          pltpu.VMEM((2, 8, _C), jnp.float32),
          pltpu.VMEM((2, 8, _C), jnp.float32),
          pltpu.SemaphoreType.DMA,
          pltpu.SemaphoreType.DMA,
          pltpu.SemaphoreType.DMA,
          pltpu.SemaphoreType.DMA,
      ],
  )(img2, rflat)

  out = pl.pallas_call(
      _depad_body,
      grid=(_NUM_ROIS // _G,),
      in_specs=[pl.BlockSpec((_G * _POOL, 8, _C), lambda i: (i, 0, 0))],
      out_specs=pl.BlockSpec((1, _G, _POOL, _POOL, _C),
                             lambda i: (0, i, 0, 0, 0)),
      out_shape=jax.ShapeDtypeStruct((1, _NUM_ROIS, _POOL, _POOL, _C),
                                     jnp.float32),
  )(staged)
  return out


# speedup vs baseline: 1.8483x; 1.3142x over previous
import jax
import jax.numpy as jnp
from jax import lax
from jax.experimental import pallas as pl
from jax.experimental.pallas import tpu as pltpu
from jax.experimental.pallas import tpu_sc as plsc

_POOL = 7
_NUM_ROIS = 300
_H = 64
_W = 64
_C = 512
_NPLANES = _NUM_ROIS * _POOL
_NUNITS = _NPLANES // 2

_info = plsc.get_sparse_core_info()
_NC = _info.num_cores
_NS = _info.num_subcores
_NW = _NC * _NS
_CV = _C // 16
_UPW = -(-_NUNITS // _NW)
_NPAIR = (_UPW + 2) // 2


def _body(img_hbm, rois_hbm, out_hbm, rois_v,
          idx0, idx1, wrow0, wrow1, rows0, rows1, out0, out1,
          gsem0, gsem1, wsem0, wsem1):
  idx = (idx0, idx1)
  wrow = (wrow0, wrow1)
  rows = (rows0, rows1)
  outv = (out0, out1)
  gsem = (gsem0, gsem1)
  wsem = (wsem0, wsem1)

  wid = lax.axis_index("s") * _NC + lax.axis_index("c")

  pltpu.sync_copy(rois_hbm, rois_v)

  lanes = lax.iota(jnp.int32, 16)
  pxv = jnp.minimum(jnp.where(lanes < 7, lanes, lanes - 7), 6)
  grpb = lanes >= 7
  live = lanes < 14

  def full16(v):
    return jnp.full((16,), v, jnp.int32)

  def unit_of(j):
    return wid + j * _NW

  def fire(u, b):
    for i in (0, 1):
      plane = 2 * u + i
      r = lax.div(plane, _POOL)
      py = plane - r * _POOL

      xv = plsc.load_gather(rois_v, [full16(4 * r)])
      yv = plsc.load_gather(rois_v, [full16(4 * r + 1)])
      wv = plsc.load_gather(rois_v, [full16(4 * r + 2)])
      hv = plsc.load_gather(rois_v, [full16(4 * r + 3)])

      ys = full16(py).astype(jnp.float32) * (hv.astype(jnp.float32)
                                             / float(_POOL))
      y0 = jnp.minimum(ys.astype(jnp.int32), hv - 1)
      y1 = jnp.minimum(y0 + 1, hv - 1)
      wy = ys - y0.astype(jnp.float32)

      xs = pxv.astype(jnp.float32) * (wv.astype(jnp.float32) / float(_POOL))
      x0 = jnp.minimum(xs.astype(jnp.int32), wv - 1)
      x1 = jnp.minimum(x0 + 1, wv - 1)
      wx = xs - x0.astype(jnp.float32)

      col = xv + jnp.where(grpb, x1, x0)
      wcol = jnp.where(live, jnp.where(grpb, wx, 1.0 - wx), 0.0)

      idx[b][pl.ds(32 * i, 16)] = (yv + y0) * _W + col
      idx[b][pl.ds(32 * i + 16, 16)] = (yv + y1) * _W + col
      wrow[b][pl.ds(32 * i, 16)] = (1.0 - wy) * wcol
      wrow[b][pl.ds(32 * i + 16, 16)] = wy * wcol

    pltpu.async_copy(img_hbm.at[idx[b]], rows[b], gsem[b])

  def blend(b):
    rv = rows[b]
    ov = outv[b]

    def do_px(px, _):
      for i in (0, 1):
        o = 32 * i
        wa = plsc.load_gather(wrow[b], [full16(px + o)])
        wb = plsc.load_gather(wrow[b], [full16(px + o + 7)])
        wc = plsc.load_gather(wrow[b], [full16(px + o + 16)])
        wd = plsc.load_gather(wrow[b], [full16(px + o + 23)])
        for v in range(_CV):
          sl = pl.ds(v * 16, 16)
          acc = (rv[px + o, sl] * wa + rv[px + o + 7, sl] * wb
                 + rv[px + o + 16, sl] * wc + rv[px + o + 23, sl] * wd)
          ov[i, px, sl] = acc
      return 0

    lax.fori_loop(0, _POOL, do_px, 0)

  def write(u, b):
    for i in (0, 1):
      plane = 2 * u + i
      r = lax.div(plane, _POOL)
      py = plane - r * _POOL
      pltpu.async_copy(outv[b].at[i, pl.ds(0, _POOL)],
                       out_hbm.at[0, r, py], wsem[b])

  def wait_gather(b):
    pltpu.make_async_copy(img_hbm.at[idx[b]], rows[b], gsem[b]).wait()

  def wait_write(b):
    for _ in (0, 1):
      pltpu.make_async_copy(outv[b].at[0, pl.ds(0, _POOL)],
                            out_hbm.at[0, 0, 0], wsem[b]).wait()

  fire(unit_of(0), 0)

  def pair_body(t, _):
    for b in (0, 1):
      j = 2 * t + b
      u = unit_of(j)
      un = unit_of(j + 1)

      @pl.when(un < _NUNITS)
      def _():
        fire(un, 1 - b)

      @pl.when(u < _NUNITS)
      def _():
        wait_gather(b)
        @pl.when(j >= 2)
        def _():
          wait_write(b)

        blend(b)
        write(u, b)

    return 0

  lax.fori_loop(0, _NPAIR, pair_body, 0)

  wait_write(0)
  wait_write(1)


_G = 10


def _depad_body(in_ref, out_ref):
  x = in_ref[...].reshape(_G, _POOL, 8, _C)
  out_ref[0] = x[:, :, :_POOL, :]


@jax.jit
def kernel(img, rois):
  img2 = img.reshape(_H * _W, _C)
  rflat = rois.reshape(-1).astype(jnp.int32)
  mesh = plsc.VectorSubcoreMesh(core_axis_name="c", subcore_axis_name="s")
  out = pl.kernel(
      _body,
      mesh=mesh,
      compiler_params=pltpu.CompilerParams(needs_layout_passes=False,
                                           use_tc_tiling_on_sc=False),
      out_type=jax.ShapeDtypeStruct((1, _NUM_ROIS, _POOL, _POOL, _C),
                                    jnp.float32),
      scratch_types=[
          pltpu.VMEM((_NUM_ROIS * 4,), jnp.int32),
          pltpu.VMEM((64,), jnp.int32),
          pltpu.VMEM((64,), jnp.int32),
          pltpu.VMEM((64,), jnp.float32),
          pltpu.VMEM((64,), jnp.float32),
          pltpu.VMEM((64, _C), jnp.float32),
          pltpu.VMEM((64, _C), jnp.float32),
          pltpu.VMEM((2, 8, _C), jnp.float32),
          pltpu.VMEM((2, 8, _C), jnp.float32),
          pltpu.SemaphoreType.DMA,
          pltpu.SemaphoreType.DMA,
          pltpu.SemaphoreType.DMA,
          pltpu.SemaphoreType.DMA,
      ],
  )(img2, rflat)
  return out


# speedup vs baseline: 1.8506x; 1.0013x over previous
import jax
import jax.numpy as jnp
from jax import lax
from jax.experimental import pallas as pl
from jax.experimental.pallas import tpu as pltpu
from jax.experimental.pallas import tpu_sc as plsc

_POOL = 7
_NUM_ROIS = 300
_H = 64
_W = 64
_C = 512
_NPLANES = _NUM_ROIS * _POOL
_NUNITS = _NPLANES // 2

_info = plsc.get_sparse_core_info()
_NC = _info.num_cores
_NS = _info.num_subcores
_NW = _NC * _NS
_CV = _C // 16
_UPW = -(-_NUNITS // _NW)
_NPAIR = (_UPW + 2) // 2


def _body(img_hbm, rois_hbm, out_hbm, rois_v,
          idx0, idx1, wrow0, wrow1, rows0, rows1, out0, out1,
          gsem0, gsem1, wsem0, wsem1):
  idx = (idx0, idx1)
  wrow = (wrow0, wrow1)
  rows = (rows0, rows1)
  outv = (out0, out1)
  gsem = (gsem0, gsem1)
  wsem = (wsem0, wsem1)

  wid = lax.axis_index("s") * _NC + lax.axis_index("c")

  pltpu.sync_copy(rois_hbm, rois_v)

  lanes = lax.iota(jnp.int32, 16)
  pxv = jnp.minimum(jnp.where(lanes < 7, lanes, lanes - 7), 6)
  grpb = lanes >= 7
  live = lanes < 14

  def full16(v):
    return jnp.full((16,), v, jnp.int32)

  def unit_of(j):
    return wid + j * _NW

  def fire(u, b):
    for i in (0, 1):
      plane = 2 * u + i
      r = lax.div(plane, _POOL)
      py = plane - r * _POOL

      xv = plsc.load_gather(rois_v, [full16(4 * r)])
      yv = plsc.load_gather(rois_v, [full16(4 * r + 1)])
      wv = plsc.load_gather(rois_v, [full16(4 * r + 2)])
      hv = plsc.load_gather(rois_v, [full16(4 * r + 3)])

      ys = full16(py).astype(jnp.float32) * (hv.astype(jnp.float32)
                                             / float(_POOL))
      y0 = jnp.minimum(ys.astype(jnp.int32), hv - 1)
      y1 = jnp.minimum(y0 + 1, hv - 1)
      wy = ys - y0.astype(jnp.float32)

      xs = pxv.astype(jnp.float32) * (wv.astype(jnp.float32) / float(_POOL))
      x0 = jnp.minimum(xs.astype(jnp.int32), wv - 1)
      x1 = jnp.minimum(x0 + 1, wv - 1)
      wx = xs - x0.astype(jnp.float32)

      col = xv + jnp.where(grpb, x1, x0)
      wcol = jnp.where(live, jnp.where(grpb, wx, 1.0 - wx), 0.0)

      idx[b][pl.ds(32 * i, 16)] = (yv + y0) * _W + col
      idx[b][pl.ds(32 * i + 16, 16)] = (yv + y1) * _W + col
      wrow[b][pl.ds(32 * i, 16)] = (1.0 - wy) * wcol
      wrow[b][pl.ds(32 * i + 16, 16)] = wy * wcol

    pltpu.async_copy(img_hbm.at[idx[b]], rows[b], gsem[b])

  def blend(b):
    rv = rows[b]
    ov = outv[b]

    def do_px(px, _):
      for i in (0, 1):
        o = 32 * i
        wa = plsc.load_gather(wrow[b], [full16(px + o)])
        wb = plsc.load_gather(wrow[b], [full16(px + o + 7)])
        wc = plsc.load_gather(wrow[b], [full16(px + o + 16)])
        wd = plsc.load_gather(wrow[b], [full16(px + o + 23)])
        for v in range(_CV):
          sl = pl.ds(v * 16, 16)
          acc = (rv[px + o, sl] * wa + rv[px + o + 7, sl] * wb
                 + rv[px + o + 16, sl] * wc + rv[px + o + 23, sl] * wd)
          ov[i, px, sl] = acc
      return 0

    lax.fori_loop(0, _POOL, do_px, 0)

  def write(u, b):
    for i in (0, 1):
      plane = 2 * u + i
      r = lax.div(plane, _POOL)
      py = plane - r * _POOL
      pltpu.async_copy(outv[b].at[i, pl.ds(0, _POOL)],
                       out_hbm.at[0, r, py], wsem[b])

  def wait_gather(b):
    pltpu.make_async_copy(img_hbm.at[idx[b]], rows[b], gsem[b]).wait()

  def wait_write(b):
    for _ in (0, 1):
      pltpu.make_async_copy(outv[b].at[0, pl.ds(0, _POOL)],
                            out_hbm.at[0, 0, 0], wsem[b]).wait()

  fire(unit_of(0), 0)

  def pair_body(t, _):
    for b in (0, 1):
      j = 2 * t + b
      u = unit_of(j)
      un = unit_of(j + 1)

      @pl.when(un < _NUNITS)
      def _():
        fire(un, 1 - b)

      @pl.when(u < _NUNITS)
      def _():
        wait_gather(b)
        @pl.when(j >= 2)
        def _():
          wait_write(b)

        blend(b)
        write(u, b)

    return 0

  lax.fori_loop(0, _NPAIR, pair_body, 0)

  wait_write(0)
  wait_write(1)


@jax.jit
def kernel(img, rois):
  img2 = img.reshape(_H * _W, _C)
  rflat = rois.reshape(-1).astype(jnp.int32)
  mesh = plsc.VectorSubcoreMesh(core_axis_name="c", subcore_axis_name="s")
  out = pl.kernel(
      _body,
      mesh=mesh,
      compiler_params=pltpu.CompilerParams(needs_layout_passes=False,
                                           use_tc_tiling_on_sc=False),
      out_type=jax.ShapeDtypeStruct((1, _NUM_ROIS, _POOL, _POOL, _C),
                                    jnp.float32),
      scratch_types=[
          pltpu.VMEM((_NUM_ROIS * 4,), jnp.int32),
          pltpu.VMEM((64,), jnp.int32),
          pltpu.VMEM((64,), jnp.int32),
          pltpu.VMEM((64,), jnp.float32),
          pltpu.VMEM((64,), jnp.float32),
          pltpu.VMEM((64, _C), jnp.float32),
          pltpu.VMEM((64, _C), jnp.float32),
          pltpu.VMEM((2, 8, _C), jnp.float32),
          pltpu.VMEM((2, 8, _C), jnp.float32),
          pltpu.SemaphoreType.DMA,
          pltpu.SemaphoreType.DMA,
          pltpu.SemaphoreType.DMA,
          pltpu.SemaphoreType.DMA,
      ],
  )(img2, rflat)
  return out
